# Initial kernel scaffold; baseline (speedup 1.0000x reference)
import jax
import jax.numpy as jnp
from jax.experimental import pallas as pl


def kernel(tokens, input_embedding, position_embedding, register_tokens):
    raise NotImplementedError("write your pallas kernel here")


# speedup vs baseline: 3.6553x; 3.6553x over previous
import functools

import jax
import jax.numpy as jnp
from jax import lax
from jax.experimental import pallas as pl
from jax.experimental.pallas import tpu as pltpu
from jax.experimental.pallas import tpu_sc as plsc

_B = 4096
_S = 200
_D = 64
_R = 4
_OUT_S = _R + _S
_NW = 32
_SEQ_PER_W = _B // _NW
_LANES = 16


def _make_kernel():
    mesh = plsc.VectorSubcoreMesh(core_axis_name="c", subcore_axis_name="s")

    @functools.partial(
        pl.kernel,
        mesh=mesh,
        out_type=jax.ShapeDtypeStruct((_B, _OUT_S, _D), jnp.float32),
        scratch_types=[
            pltpu.VMEM((_SEQ_PER_W, _S), jnp.int32),
            pltpu.VMEM((_S, _D), jnp.float32),
            pltpu.VMEM((_S, 2 * _D), jnp.float32),
            pltpu.VMEM((_OUT_S, _D), jnp.float32),
            pltpu.SemaphoreType.DMA,
        ],
    )
    def emb_kernel(tok_hbm, table_hbm, pos_hbm, reg_hbm, out_hbm,
                   idx_v, pos_v, gat_v, buf_v, gsem):
        wid = lax.axis_index("s") * 2 + lax.axis_index("c")
        base = wid * _SEQ_PER_W
        pltpu.sync_copy(tok_hbm.at[pl.ds(base, _SEQ_PER_W)], idx_v)
        pltpu.sync_copy(pos_hbm, pos_v)
        pltpu.sync_copy(reg_hbm, buf_v.at[pl.ds(0, _R)])

        def seq_body(i, carry):
            c0 = pltpu.async_copy(
                table_hbm.at[idx_v.at[i, pl.ds(0, 128)]],
                gat_v.at[pl.ds(0, 128)], gsem)
            c1 = pltpu.async_copy(
                table_hbm.at[idx_v.at[i, pl.ds(128, _S - 128)]],
                gat_v.at[pl.ds(128, _S - 128)], gsem)
            c0.wait()
            c1.wait()

            def add_row(r, c2):
                for c in range(_D // _LANES):
                    sl = pl.ds(c * _LANES, _LANES)
                    buf_v[r + _R, sl] = gat_v[r, sl] + pos_v[r, sl]
                return c2

            lax.fori_loop(0, _S, add_row, 0)
            pltpu.sync_copy(buf_v, out_hbm.at[base + i])
            return carry

        lax.fori_loop(0, _SEQ_PER_W, seq_body, 0)

    return emb_kernel


_EMB_KERNEL = _make_kernel()


def _restage_table(input_embedding):
    return jnp.pad(input_embedding, ((0, 0), (0, 2 * _D - _D)))


@jax.jit
def kernel(tokens, input_embedding, position_embedding, register_tokens):
    table = _restage_table(input_embedding)
    return _EMB_KERNEL(tokens, table, position_embedding, register_tokens)


# speedup vs baseline: 3.7489x; 1.0256x over previous
import functools

import jax
import jax.numpy as jnp
from jax import lax
from jax.experimental import pallas as pl
from jax.experimental.pallas import tpu as pltpu
from jax.experimental.pallas import tpu_sc as plsc

_B = 4096
_S = 200
_D = 64
_R = 4
_OUT_S = _R + _S
_NW = 32
_SEQ_PER_W = _B // _NW
_LANES = 16
_C0 = 128
_C1 = _S - _C0


def _make_kernel():
    mesh = plsc.VectorSubcoreMesh(core_axis_name="c", subcore_axis_name="s")

    @functools.partial(
        pl.kernel,
        mesh=mesh,
        out_type=jax.ShapeDtypeStruct((_B, _OUT_S, _D), jnp.float32),
        scratch_types=[
            pltpu.VMEM((_S // 2, 2 * _D), jnp.float32),
            pltpu.VMEM((256,), jnp.int32),
            pltpu.VMEM((256,), jnp.int32),
            pltpu.VMEM((_S, 2 * _D), jnp.float32),
            pltpu.VMEM((_S, 2 * _D), jnp.float32),
            pltpu.VMEM((_OUT_S, _D), jnp.float32),
            pltpu.VMEM((_OUT_S, _D), jnp.float32),
            pltpu.SemaphoreType.DMA,
            pltpu.SemaphoreType.DMA,
            pltpu.SemaphoreType.DMA,
            pltpu.SemaphoreType.DMA,
            pltpu.SemaphoreType.DMA,
            pltpu.SemaphoreType.DMA,
        ],
    )
    def emb_kernel(tok_hbm, table_hbm, pos_hbm, reg_hbm, out_hbm,
                   pos_v, idx0, idx1, gat0, gat1, buf0, buf1,
                   gsem0, gsem1, osem0, osem1, isem0, isem1):
        wid = lax.axis_index("s") * 2 + lax.axis_index("c")
        base = wid * _SEQ_PER_W
        pltpu.sync_copy(pos_hbm, pos_v)
        pltpu.sync_copy(reg_hbm, buf0.at[pl.ds(0, _R)])
        pltpu.sync_copy(reg_hbm, buf1.at[pl.ds(0, _R)])

        def start_idx(i, idx, isem):
            pltpu.async_copy(tok_hbm.at[pl.ds((base + i) * 256, 256)], idx,
                             isem)

        def drain_idx(idx, isem):
            pltpu.make_async_copy(tok_hbm.at[pl.ds(0, 256)], idx, isem).wait()

        def start_gather(idx, gat, gsem):
            pltpu.async_copy(table_hbm.at[idx.at[pl.ds(0, _C0)]],
                             gat.at[pl.ds(0, _C0)], gsem)
            pltpu.async_copy(table_hbm.at[idx.at[pl.ds(_C0, _C1)]],
                             gat.at[pl.ds(_C0, _C1)], gsem)

        def drain_gather(gat, gsem):
            pltpu.make_async_copy(table_hbm.at[pl.ds(0, _S)], gat, gsem).wait()

        start_idx(0, idx0, isem0)
        start_idx(1, idx1, isem1)
        drain_idx(idx0, isem0)
        drain_idx(idx1, isem1)
        start_gather(idx0, gat0, gsem0)
        start_gather(idx1, gat1, gsem1)

        def seq_body(j, carry):
            for s, (idx, gat, buf, gsem, osem, isem) in enumerate((
                    (idx0, gat0, buf0, gsem0, osem0, isem0),
                    (idx1, gat1, buf1, gsem1, osem1, isem1))):
                i = 2 * j + s
                b = base + i
                drain_gather(gat, gsem)

                @pl.when(i + 2 < _SEQ_PER_W)
                def _():
                    start_idx(i + 2, idx, isem)

                @pl.when(j > 0)
                def _():
                    pltpu.make_async_copy(buf, out_hbm.at[b], osem).wait()

                def add_rows(r2, c2):
                    for c in range(_D // _LANES):
                        sl = pl.ds(c * _LANES, _LANES)
                        buf[2 * r2 + _R, sl] = (
                            gat[2 * r2, sl]
                            + pos_v[r2, pl.ds(c * _LANES, _LANES)])
                        buf[2 * r2 + 1 + _R, sl] = (
                            gat[2 * r2 + 1, sl]
                            + pos_v[r2, pl.ds(_D + c * _LANES, _LANES)])
                    return c2

                lax.fori_loop(0, _S // 2, add_rows, 0)
                pltpu.async_copy(buf, out_hbm.at[b], osem)

                @pl.when(i + 2 < _SEQ_PER_W)
                def _():
                    drain_idx(idx, isem)
                    start_gather(idx, gat, gsem)
            return carry

        lax.fori_loop(0, _SEQ_PER_W // 2, seq_body, 0)
        pltpu.make_async_copy(buf0, out_hbm.at[base], osem0).wait()
        pltpu.make_async_copy(buf1, out_hbm.at[base], osem1).wait()

    return emb_kernel


_EMB_KERNEL = _make_kernel()


def _restage_table(input_embedding):
    return jnp.pad(input_embedding, ((0, 0), (0, 2 * _D - _D)))


@jax.jit
def kernel(tokens, input_embedding, position_embedding, register_tokens):
    table = _restage_table(input_embedding)
    pos2 = position_embedding.reshape(_S // 2, 2 * _D)
    tok_flat = jnp.pad(tokens, ((0, 0), (0, 256 - _S))).reshape(-1)
    return _EMB_KERNEL(tok_flat, table, pos2, register_tokens)


# speedup vs baseline: 3.7582x; 1.0025x over previous
import functools

import jax
import jax.numpy as jnp
from jax import lax
from jax.experimental import pallas as pl
from jax.experimental.pallas import tpu as pltpu
from jax.experimental.pallas import tpu_sc as plsc

_B = 4096
_S = 200
_D = 64
_R = 4
_OUT_S = _R + _S
_NW = 32
_SEQ_PER_W = _B // _NW
_LANES = 16
_C0 = 128
_C1 = _S - _C0


def _make_kernel():
    mesh = plsc.VectorSubcoreMesh(core_axis_name="c", subcore_axis_name="s")

    @functools.partial(
        pl.kernel,
        mesh=mesh,
        compiler_params=pltpu.CompilerParams(use_tc_tiling_on_sc=True),
        out_type=jax.ShapeDtypeStruct((_B, _OUT_S, _D), jnp.float32),
        scratch_types=[
            pltpu.VMEM((_S // 2, 2 * _D), jnp.float32),
            pltpu.VMEM((256,), jnp.int32),
            pltpu.VMEM((256,), jnp.int32),
            pltpu.VMEM((_S, 2 * _D), jnp.float32),
            pltpu.VMEM((_S, 2 * _D), jnp.float32),
            pltpu.VMEM((_OUT_S, _D), jnp.float32),
            pltpu.VMEM((_OUT_S, _D), jnp.float32),
            pltpu.SemaphoreType.DMA,
            pltpu.SemaphoreType.DMA,
            pltpu.SemaphoreType.DMA,
            pltpu.SemaphoreType.DMA,
            pltpu.SemaphoreType.DMA,
            pltpu.SemaphoreType.DMA,
        ],
    )
    def emb_kernel(tok_hbm, table_hbm, pos_hbm, reg_hbm, out_hbm,
                   pos_v, idx0, idx1, gat0, gat1, buf0, buf1,
                   gsem0, gsem1, osem0, osem1, isem0, isem1):
        wid = lax.axis_index("s") * 2 + lax.axis_index("c")
        base = wid * _SEQ_PER_W
        pltpu.sync_copy(pos_hbm, pos_v)
        pltpu.sync_copy(reg_hbm, buf0.at[pl.ds(0, _R)])
        pltpu.sync_copy(reg_hbm, buf1.at[pl.ds(0, _R)])

        def start_idx(i, idx, isem):
            pltpu.async_copy(tok_hbm.at[pl.ds((base + i) * 256, 256)], idx,
                             isem)

        def drain_idx(idx, isem):
            pltpu.make_async_copy(tok_hbm.at[pl.ds(0, 256)], idx, isem).wait()

        def start_gather(idx, gat, gsem):
            pltpu.async_copy(table_hbm.at[idx.at[pl.ds(0, _C0)]],
                             gat.at[pl.ds(0, _C0)], gsem)
            pltpu.async_copy(table_hbm.at[idx.at[pl.ds(_C0, _C1)]],
                             gat.at[pl.ds(_C0, _C1)], gsem)

        def drain_gather(gat, gsem):
            pltpu.make_async_copy(table_hbm.at[pl.ds(0, _S)], gat, gsem).wait()

        start_idx(0, idx0, isem0)
        start_idx(1, idx1, isem1)
        drain_idx(idx0, isem0)
        drain_idx(idx1, isem1)
        start_gather(idx0, gat0, gsem0)
        start_gather(idx1, gat1, gsem1)

        def seq_body(j, carry):
            for s, (idx, gat, buf, gsem, osem, isem) in enumerate((
                    (idx0, gat0, buf0, gsem0, osem0, isem0),
                    (idx1, gat1, buf1, gsem1, osem1, isem1))):
                i = 2 * j + s
                b = base + i
                drain_gather(gat, gsem)

                @pl.when(i + 2 < _SEQ_PER_W)
                def _():
                    start_idx(i + 2, idx, isem)

                @pl.when(j > 0)
                def _():
                    pltpu.make_async_copy(buf, out_hbm.at[b], osem).wait()

                def add_rows(r2, c2):
                    for c in range(_D // _LANES):
                        sl = pl.ds(c * _LANES, _LANES)
                        buf[2 * r2 + _R, sl] = (
                            gat[2 * r2, sl]
                            + pos_v[r2, pl.ds(c * _LANES, _LANES)])
                        buf[2 * r2 + 1 + _R, sl] = (
                            gat[2 * r2 + 1, sl]
                            + pos_v[r2, pl.ds(_D + c * _LANES, _LANES)])
                    return c2

                lax.fori_loop(0, _S // 2, add_rows, 0)
                pltpu.async_copy(buf, out_hbm.at[b], osem)

                @pl.when(i + 2 < _SEQ_PER_W)
                def _():
                    drain_idx(idx, isem)
                    start_gather(idx, gat, gsem)
            return carry

        lax.fori_loop(0, _SEQ_PER_W // 2, seq_body, 0)
        pltpu.make_async_copy(buf0, out_hbm.at[base], osem0).wait()
        pltpu.make_async_copy(buf1, out_hbm.at[base], osem1).wait()

    return emb_kernel


_EMB_KERNEL = _make_kernel()


def _restage_table(input_embedding):
    return jnp.pad(input_embedding, ((0, 0), (0, 2 * _D - _D)))


@jax.jit
def kernel(tokens, input_embedding, position_embedding, register_tokens):
    table = _restage_table(input_embedding)
    pos2 = position_embedding.reshape(_S // 2, 2 * _D)
    tok_flat = jnp.pad(tokens, ((0, 0), (0, 256 - _S))).reshape(-1)
    return _EMB_KERNEL(tok_flat, table, pos2, register_tokens)


# speedup vs baseline: 4.3601x; 1.1602x over previous
import functools

import jax
import jax.numpy as jnp
from jax import lax
from jax.experimental import pallas as pl
from jax.experimental.pallas import tpu as pltpu
from jax.experimental.pallas import tpu_sc as plsc

_B = 4096
_S = 200
_D = 64
_R = 4
_OUT_S = _R + _S
_NW = 32
_SEQ_PER_W = _B // _NW
_LANES = 16
_C0 = 128
_C1 = _S - _C0


def _make_kernel():
    mesh = plsc.VectorSubcoreMesh(core_axis_name="c", subcore_axis_name="s")

    @functools.partial(
        pl.kernel,
        mesh=mesh,
        compiler_params=pltpu.CompilerParams(use_tc_tiling_on_sc=False),
        out_type=jax.ShapeDtypeStruct((_B, _OUT_S, _D), jnp.float32),
        scratch_types=[
            pltpu.VMEM((_S // 2, 2 * _D), jnp.float32),
            pltpu.VMEM((256,), jnp.int32),
            pltpu.VMEM((256,), jnp.int32),
            pltpu.VMEM((_S, _D), jnp.float32),
            pltpu.VMEM((_S, _D), jnp.float32),
            pltpu.VMEM((_OUT_S, _D), jnp.float32),
            pltpu.VMEM((_OUT_S, _D), jnp.float32),
            pltpu.SemaphoreType.DMA,
            pltpu.SemaphoreType.DMA,
            pltpu.SemaphoreType.DMA,
            pltpu.SemaphoreType.DMA,
            pltpu.SemaphoreType.DMA,
            pltpu.SemaphoreType.DMA,
        ],
    )
    def emb_kernel(tok_hbm, table_hbm, pos_hbm, reg_hbm, out_hbm,
                   pos_v, idx0, idx1, gat0, gat1, buf0, buf1,
                   gsem0, gsem1, osem0, osem1, isem0, isem1):
        wid = lax.axis_index("s") * 2 + lax.axis_index("c")
        base = wid * _SEQ_PER_W
        pltpu.sync_copy(pos_hbm, pos_v)
        pltpu.sync_copy(reg_hbm, buf0.at[pl.ds(0, _R)])
        pltpu.sync_copy(reg_hbm, buf1.at[pl.ds(0, _R)])

        def start_idx(i, idx, isem):
            pltpu.async_copy(tok_hbm.at[pl.ds((base + i) * 256, 256)], idx,
                             isem)

        def drain_idx(idx, isem):
            pltpu.make_async_copy(tok_hbm.at[pl.ds(0, 256)], idx, isem).wait()

        def start_gather(idx, gat, gsem):
            pltpu.async_copy(table_hbm.at[idx.at[pl.ds(0, _C0)]],
                             gat.at[pl.ds(0, _C0)], gsem)
            pltpu.async_copy(table_hbm.at[idx.at[pl.ds(_C0, _C1)]],
                             gat.at[pl.ds(_C0, _C1)], gsem)

        def drain_gather(gat, gsem):
            pltpu.make_async_copy(table_hbm.at[pl.ds(0, _S)], gat, gsem).wait()

        start_idx(0, idx0, isem0)
        start_idx(1, idx1, isem1)
        drain_idx(idx0, isem0)
        drain_idx(idx1, isem1)
        start_gather(idx0, gat0, gsem0)
        start_gather(idx1, gat1, gsem1)

        def seq_body(j, carry):
            for s, (idx, gat, buf, gsem, osem, isem) in enumerate((
                    (idx0, gat0, buf0, gsem0, osem0, isem0),
                    (idx1, gat1, buf1, gsem1, osem1, isem1))):
                i = 2 * j + s
                b = base + i
                drain_gather(gat, gsem)

                @pl.when(i + 2 < _SEQ_PER_W)
                def _():
                    start_idx(i + 2, idx, isem)

                @pl.when(j > 0)
                def _():
                    pltpu.make_async_copy(buf, out_hbm.at[b], osem).wait()

                def add_rows(r2, c2):
                    for c in range(_D // _LANES):
                        sl = pl.ds(c * _LANES, _LANES)
                        buf[2 * r2 + _R, sl] = (
                            gat[2 * r2, sl]
                            + pos_v[r2, pl.ds(c * _LANES, _LANES)])
                        buf[2 * r2 + 1 + _R, sl] = (
                            gat[2 * r2 + 1, sl]
                            + pos_v[r2, pl.ds(_D + c * _LANES, _LANES)])
                    return c2

                lax.fori_loop(0, _S // 2, add_rows, 0)
                pltpu.async_copy(buf, out_hbm.at[b], osem)

                @pl.when(i + 2 < _SEQ_PER_W)
                def _():
                    drain_idx(idx, isem)
                    start_gather(idx, gat, gsem)
            return carry

        lax.fori_loop(0, _SEQ_PER_W // 2, seq_body, 0)
        pltpu.make_async_copy(buf0, out_hbm.at[base], osem0).wait()
        pltpu.make_async_copy(buf1, out_hbm.at[base], osem1).wait()

    return emb_kernel


_EMB_KERNEL = _make_kernel()


@jax.jit
def kernel(tokens, input_embedding, position_embedding, register_tokens):
    pos2 = position_embedding.reshape(_S // 2, 2 * _D)
    tok_flat = jnp.pad(tokens, ((0, 0), (0, 256 - _S))).reshape(-1)
    return _EMB_KERNEL(tok_flat, input_embedding, pos2, register_tokens)


# speedup vs baseline: 4.3617x; 1.0004x over previous
import functools

import jax
import jax.numpy as jnp
from jax import lax
from jax.experimental import pallas as pl
from jax.experimental.pallas import tpu as pltpu
from jax.experimental.pallas import tpu_sc as plsc

_B = 4096
_S = 200
_D = 64
_R = 4
_OUT_S = _R + _S
_NW = 32
_SEQ_PER_W = _B // _NW
_LANES = 16
_C0 = 128
_C1 = _S - _C0


def _make_kernel():
    mesh = plsc.VectorSubcoreMesh(core_axis_name="c", subcore_axis_name="s")

    @functools.partial(
        pl.kernel,
        mesh=mesh,
        compiler_params=pltpu.CompilerParams(use_tc_tiling_on_sc=False),
        out_type=jax.ShapeDtypeStruct((_B * _OUT_S // 2, 2 * _D), jnp.float32),
        scratch_types=[
            pltpu.VMEM((_S // 2, 2 * _D), jnp.float32),
            pltpu.VMEM((256,), jnp.int32),
            pltpu.VMEM((256,), jnp.int32),
            pltpu.VMEM((_S, _D), jnp.float32),
            pltpu.VMEM((_S, _D), jnp.float32),
            pltpu.VMEM((_OUT_S // 2, 2 * _D), jnp.float32),
            pltpu.VMEM((_OUT_S // 2, 2 * _D), jnp.float32),
            pltpu.SemaphoreType.DMA,
            pltpu.SemaphoreType.DMA,
            pltpu.SemaphoreType.DMA,
            pltpu.SemaphoreType.DMA,
            pltpu.SemaphoreType.DMA,
            pltpu.SemaphoreType.DMA,
        ],
    )
    def emb_kernel(tok_hbm, table_hbm, pos_hbm, reg_hbm, out_hbm,
                   pos_v, idx0, idx1, gat0, gat1, buf0, buf1,
                   gsem0, gsem1, osem0, osem1, isem0, isem1):
        wid = lax.axis_index("s") * 2 + lax.axis_index("c")
        base = wid * _SEQ_PER_W
        pltpu.sync_copy(pos_hbm, pos_v)
        pltpu.sync_copy(reg_hbm, buf0.at[pl.ds(0, _R // 2)])
        pltpu.sync_copy(reg_hbm, buf1.at[pl.ds(0, _R // 2)])

        def start_idx(i, idx, isem):
            pltpu.async_copy(tok_hbm.at[pl.ds((base + i) * 256, 256)], idx,
                             isem)

        def drain_idx(idx, isem):
            pltpu.make_async_copy(tok_hbm.at[pl.ds(0, 256)], idx, isem).wait()

        def start_gather(idx, gat, gsem):
            pltpu.async_copy(table_hbm.at[idx.at[pl.ds(0, _C0)]],
                             gat.at[pl.ds(0, _C0)], gsem)
            pltpu.async_copy(table_hbm.at[idx.at[pl.ds(_C0, _C1)]],
                             gat.at[pl.ds(_C0, _C1)], gsem)

        def drain_gather(gat, gsem):
            pltpu.make_async_copy(table_hbm.at[pl.ds(0, _S)], gat, gsem).wait()

        start_idx(0, idx0, isem0)
        start_idx(1, idx1, isem1)
        drain_idx(idx0, isem0)
        drain_idx(idx1, isem1)
        start_gather(idx0, gat0, gsem0)
        start_gather(idx1, gat1, gsem1)

        def seq_body(j, carry):
            for s, (idx, gat, buf, gsem, osem, isem) in enumerate((
                    (idx0, gat0, buf0, gsem0, osem0, isem0),
                    (idx1, gat1, buf1, gsem1, osem1, isem1))):
                i = 2 * j + s
                b = base + i
                drain_gather(gat, gsem)

                @pl.when(i + 2 < _SEQ_PER_W)
                def _():
                    start_idx(i + 2, idx, isem)

                @pl.when(j > 0)
                def _():
                    pltpu.make_async_copy(
                        buf, out_hbm.at[pl.ds(0, _OUT_S // 2)], osem).wait()

                def add_rows(r2, c2):
                    for c in range(_D // _LANES):
                        sl = pl.ds(c * _LANES, _LANES)
                        buf[r2 + _R // 2, sl] = (
                            gat[2 * r2, sl]
                            + pos_v[r2, pl.ds(c * _LANES, _LANES)])
                        buf[r2 + _R // 2, pl.ds(_D + c * _LANES, _LANES)] = (
                            gat[2 * r2 + 1, sl]
                            + pos_v[r2, pl.ds(_D + c * _LANES, _LANES)])
                    return c2

                lax.fori_loop(0, _S // 2, add_rows, 0)
                pltpu.async_copy(
                    buf, out_hbm.at[pl.ds(b * (_OUT_S // 2), _OUT_S // 2)],
                    osem)

                @pl.when(i + 2 < _SEQ_PER_W)
                def _():
                    drain_idx(idx, isem)
                    start_gather(idx, gat, gsem)
            return carry

        lax.fori_loop(0, _SEQ_PER_W // 2, seq_body, 0)
        pltpu.make_async_copy(buf0, out_hbm.at[pl.ds(0, _OUT_S // 2)],
                              osem0).wait()
        pltpu.make_async_copy(buf1, out_hbm.at[pl.ds(0, _OUT_S // 2)],
                              osem1).wait()

    return emb_kernel


_EMB_KERNEL = _make_kernel()


@jax.jit
def kernel(tokens, input_embedding, position_embedding, register_tokens):
    pos2 = position_embedding.reshape(_S // 2, 2 * _D)
    reg2 = register_tokens.reshape(_R // 2, 2 * _D)
    tok_flat = jnp.pad(tokens, ((0, 0), (0, 256 - _S))).reshape(-1)
    out2d = _EMB_KERNEL(tok_flat, input_embedding, pos2, reg2)
    return out2d.reshape(_B, _OUT_S, _D)
